# baseline (speedup 1.0000x reference)
import numpy as np
import jax
import jax.numpy as jnp
from jax.experimental import pallas as pl
from jax.experimental.pallas import tpu as pltpu

_NUM_MASKS = 2
_MAX_WIDTH = 40
_ALIGN = 128
_WMAX = 512


def _span_list(B, T):
    rng = np.random.RandomState(0)
    spans = []
    for b in range(B):
        for _ in range(_NUM_MASKS):
            width = int(rng.randint(1, _MAX_WIDTH + 1))
            if T - width <= 0:
                continue
            start = int(rng.randint(0, T - width))
            spans.append((b, start, width))
    return spans


def _merged_windows(B, T):
    spans = _span_list(B, T)
    per_b = {}
    for b, s, w in spans:
        a0 = (s // _ALIGN) * _ALIGN
        a1 = min(T, -(-(s + w) // _ALIGN) * _ALIGN)
        per_b.setdefault(b, []).append((a0, a1))
    windows = []
    for b in sorted(per_b):
        ivs = sorted(per_b[b])
        merged = [list(ivs[0])]
        for a0, a1 in ivs[1:]:
            if a0 <= merged[-1][1]:
                merged[-1][1] = max(merged[-1][1], a1)
            else:
                merged.append([a0, a1])
        windows.extend((b, a0, a1 - a0) for a0, a1 in merged)
    mask = np.ones((len(windows), 1, _WMAX), dtype=np.float32)
    for i, (b, a0, wlen) in enumerate(windows):
        for bb, s, w in spans:
            if bb != b:
                continue
            lo = max(0, s - a0)
            hi = min(wlen, s + w - a0)
            if hi > lo:
                mask[i, 0, lo:hi] = 0.0
    return windows, mask


def kernel(x):
    B, C, T = x.shape
    windows, mask_np = _merged_windows(B, T)
    nw = len(windows)
    mask = jnp.asarray(mask_np)

    def body(m_ref, x_ref, o_ref, wbuf, in_sems, out_sems):
        loads = []
        for i, (b, a0, wlen) in enumerate(windows):
            c = pltpu.make_async_copy(
                x_ref.at[pl.ds(b, 1), :, pl.ds(a0, wlen)],
                wbuf.at[pl.ds(i, 1), :, pl.ds(0, wlen)],
                in_sems.at[i],
            )
            c.start()
            loads.append(c)
        for c in loads:
            c.wait()
        wbuf[...] = wbuf[...] * m_ref[...]
        stores = []
        for i, (b, a0, wlen) in enumerate(windows):
            c = pltpu.make_async_copy(
                wbuf.at[pl.ds(i, 1), :, pl.ds(0, wlen)],
                o_ref.at[pl.ds(b, 1), :, pl.ds(a0, wlen)],
                out_sems.at[i],
            )
            c.start()
            stores.append(c)
        for c in stores:
            c.wait()

    return pl.pallas_call(
        body,
        in_specs=[
            pl.BlockSpec((nw, 1, _WMAX), lambda: (0, 0, 0)),
            pl.BlockSpec(memory_space=pl.ANY),
        ],
        out_specs=pl.BlockSpec(memory_space=pl.ANY),
        out_shape=jax.ShapeDtypeStruct((B, C, T), x.dtype),
        scratch_shapes=[
            pltpu.VMEM((nw, C, _WMAX), x.dtype),
            pltpu.SemaphoreType.DMA((nw,)),
            pltpu.SemaphoreType.DMA((nw,)),
        ],
        input_output_aliases={1: 0},
    )(mask, x)


# speedup vs baseline: 1.0226x; 1.0226x over previous
import numpy as np
import jax
import jax.numpy as jnp
from jax.experimental import pallas as pl
from jax.experimental.pallas import tpu as pltpu

_NUM_MASKS = 2
_MAX_WIDTH = 40


def _span_mask(B, T):
    rng = np.random.RandomState(0)
    mask = np.ones((B, 1, T), dtype=np.float32)
    for b in range(B):
        for _ in range(_NUM_MASKS):
            width = int(rng.randint(1, _MAX_WIDTH + 1))
            if T - width <= 0:
                continue
            start = int(rng.randint(0, T - width))
            mask[b, 0, start:start + width] = 0.0
    return mask


def _mask_mul(x_ref, m_ref, o_ref):
    o_ref[...] = x_ref[...] * m_ref[...]


def kernel(x):
    B, C, T = x.shape
    mask = jnp.asarray(_span_mask(B, T))

    Ct = 64
    grid = (B, C // Ct)
    return pl.pallas_call(
        _mask_mul,
        grid=grid,
        in_specs=[
            pl.BlockSpec((1, Ct, T), lambda b, c: (b, c, 0)),
            pl.BlockSpec((1, 1, T), lambda b, c: (b, 0, 0)),
        ],
        out_specs=pl.BlockSpec((1, Ct, T), lambda b, c: (b, c, 0)),
        out_shape=jax.ShapeDtypeStruct((B, C, T), x.dtype),
    )(x, mask)


# speedup vs baseline: 1.0434x; 1.0203x over previous
import numpy as np
import jax
import jax.numpy as jnp
from jax.experimental import pallas as pl
from jax.experimental.pallas import tpu as pltpu

_NUM_MASKS = 2
_MAX_WIDTH = 40


def _span_mask(B, T):
    rng = np.random.RandomState(0)
    mask = np.ones((B, 1, T), dtype=np.float32)
    for b in range(B):
        for _ in range(_NUM_MASKS):
            width = int(rng.randint(1, _MAX_WIDTH + 1))
            if T - width <= 0:
                continue
            start = int(rng.randint(0, T - width))
            mask[b, 0, start:start + width] = 0.0
    return mask


def _mask_mul(x_ref, m_ref, o_ref):
    o_ref[...] = x_ref[...] * m_ref[...]


def kernel(x):
    B, C, T = x.shape
    mask = jnp.asarray(_span_mask(B, T))

    Ct = 256
    grid = (B, C // Ct)
    return pl.pallas_call(
        _mask_mul,
        grid=grid,
        in_specs=[
            pl.BlockSpec((1, Ct, T), lambda b, c: (b, c, 0)),
            pl.BlockSpec((1, 1, T), lambda b, c: (b, 0, 0)),
        ],
        out_specs=pl.BlockSpec((1, Ct, T), lambda b, c: (b, c, 0)),
        out_shape=jax.ShapeDtypeStruct((B, C, T), x.dtype),
        compiler_params=pltpu.CompilerParams(vmem_limit_bytes=100 * 1024 * 1024),
    )(x, mask)
